# baseline (speedup 1.0000x reference)
import functools

import jax
import jax.numpy as jnp
from jax import lax
from jax.experimental import pallas as pl
from jax.experimental.pallas import tpu as pltpu
from jax.experimental.pallas import tpu_sc as plsc

NC = 2
NS = 16
L = 16
NW = NC * NS
U = 5


def _make_kernel(n_atoms, n_pairs, chunk):
  n_chunks_total = n_pairs // chunk
  k_steps = -(-n_chunks_total // NW)
  k_steps += k_steps % 2
  slc = (n_atoms // NS) // (L * U) * (L * U)
  tail = n_atoms - NS * slc
  assert n_pairs % chunk == 0 and chunk % (L * U) == 0 and chunk % 128 == 0
  assert slc % 8 == 0 and tail % (L * U) == 0 and k_steps >= 4

  mesh = plsc.VectorSubcoreMesh(
      core_axis_name="c", subcore_axis_name="s",
      num_cores=NC, num_subcores=NS)

  @functools.partial(
      pl.kernel,
      out_type=jax.ShapeDtypeStruct((n_pairs,), jnp.float32),
      mesh=mesh,
      compiler_params=pltpu.CompilerParams(needs_layout_passes=False),
      scratch_types=[
          pltpu.VMEM_SHARED((n_atoms,), jnp.float32),
          pltpu.VMEM_SHARED((NS * L,), jnp.float32),
          pltpu.VMEM((slc + tail,), jnp.int32),
          pltpu.VMEM((slc + tail,), jnp.float32),
          pltpu.VMEM((L,), jnp.float32),
          pltpu.VMEM((NS * L,), jnp.float32),
          pltpu.VMEM((chunk,), jnp.int32),
          pltpu.VMEM((chunk,), jnp.int32),
          pltpu.VMEM((chunk,), jnp.int32),
          pltpu.VMEM((chunk,), jnp.int32),
          pltpu.VMEM((chunk,), jnp.float32),
          pltpu.VMEM((chunk,), jnp.float32),
          pltpu.VMEM((chunk,), jnp.float32),
          pltpu.VMEM((chunk,), jnp.float32),
          pltpu.VMEM((chunk,), jnp.float32),
          pltpu.VMEM((chunk,), jnp.float32),
          pltpu.SemaphoreType.DMA((2,)),
          pltpu.SemaphoreType.DMA((2,)),
          pltpu.SemaphoreType.DMA((2,)),
      ],
  )
  def k(elem_hbm, nbr_hbm, base_hbm, out_hbm,
        shared_q, shared_part, cv_in, cv_out, base_v, part_v,
        idx_i0, idx_i1, idx_j0, idx_j1, qi0, qi1, qj0, qj1, out0, out1,
        sem_in, sem_g, sem_out):
    wid = lax.axis_index("s") * NC + lax.axis_index("c")
    sid = lax.axis_index("s")
    idx_i_b = (idx_i0, idx_i1)
    idx_j_b = (idx_j0, idx_j1)
    qi_b = (qi0, qi1)
    qj_b = (qj0, qj1)
    out_b = (out0, out1)

    pltpu.sync_copy(base_hbm, base_v)
    my_off = pl.multiple_of(sid * slc, 8)

    def convert(lo, n):
      def body(i, acc):
        base_off = lo + i * (L * U)
        sls = [pl.ds(base_off + u * L, L) for u in range(U)]
        es = [cv_in[sl] for sl in sls]
        cs = [plsc.load_gather(base_v, [e]) for e in es]
        for sl, c in zip(sls, cs):
          cv_out[sl] = c
        for c in cs:
          acc = acc + c
        return acc
      return lax.fori_loop(0, n // (L * U), body,
                           jnp.zeros((L,), jnp.float32))

    pltpu.sync_copy(elem_hbm.at[pl.ds(my_off, slc)],
                    cv_in.at[pl.ds(0, slc)])
    acc = convert(0, slc)

    @pl.when(sid == NS - 1)
    def _():
      pltpu.sync_copy(elem_hbm.at[pl.ds(NS * slc, tail)],
                      cv_in.at[pl.ds(slc, tail)])

    acc = lax.cond(
        sid == NS - 1, lambda: acc + convert(slc, tail), lambda: acc)

    pltpu.sync_copy(cv_out.at[pl.ds(0, slc)],
                    shared_q.at[pl.ds(my_off, slc)])

    @pl.when(sid == NS - 1)
    def _():
      pltpu.sync_copy(cv_out.at[pl.ds(slc, tail)],
                      shared_q.at[pl.ds(NS * slc, tail)])

    cv_out[pl.ds(0, L)] = acc
    pltpu.sync_copy(cv_out.at[pl.ds(0, L)],
                    shared_part.at[pl.ds(pl.multiple_of(sid * L, 8), L)])
    plsc.subcore_barrier()
    pltpu.sync_copy(shared_part, part_v)
    tot = jnp.zeros((L,), jnp.float32)
    for t in range(NS):
      tot = tot + part_v[pl.ds(t * L, L)]
    m = jnp.sum(tot) * (1.0 / float(n_atoms))
    m_vec = jnp.full((L,), m, jnp.float32)

    def chunk_off(k_):
      return pl.multiple_of((wid + NW * k_) * chunk, chunk)

    def valid(k_):
      return wid + NW * k_ < n_chunks_total

    def start_in(k_, b):
      off = chunk_off(k_)
      pltpu.async_copy(nbr_hbm.at[0, pl.ds(off, chunk)], idx_i_b[b],
                       sem_in.at[b])
      pltpu.async_copy(nbr_hbm.at[1, pl.ds(off, chunk)], idx_j_b[b],
                       sem_in.at[b])

    def wait_in(k_, b):
      off = chunk_off(k_)
      pltpu.make_async_copy(nbr_hbm.at[0, pl.ds(off, chunk)], idx_i_b[b],
                            sem_in.at[b]).wait()
      pltpu.make_async_copy(nbr_hbm.at[1, pl.ds(off, chunk)], idx_j_b[b],
                            sem_in.at[b]).wait()

    def start_gather(b):
      pltpu.async_copy(shared_q.at[idx_i_b[b]], qi_b[b], sem_g.at[b])
      pltpu.async_copy(shared_q.at[idx_j_b[b]], qj_b[b], sem_g.at[b])

    def wait_gather(b):
      pltpu.make_async_copy(shared_q.at[idx_i_b[b]], qi_b[b],
                            sem_g.at[b]).wait()
      pltpu.make_async_copy(shared_q.at[idx_j_b[b]], qj_b[b],
                            sem_g.at[b]).wait()

    def start_out(k_, b):
      off = chunk_off(k_)
      pltpu.async_copy(out_b[b], out_hbm.at[pl.ds(off, chunk)],
                       sem_out.at[b])

    def wait_out(k_, b):
      off = chunk_off(k_)
      pltpu.make_async_copy(out_b[b], out_hbm.at[pl.ds(off, chunk)],
                            sem_out.at[b]).wait()

    def compute(b):
      qi = qi_b[b]
      qj = qj_b[b]
      ob = out_b[b]

      def body(t, _):
        base_t = t * (L * U)
        sls = [pl.ds(base_t + u * L, L) for u in range(U)]
        qis = [qi[sl] for sl in sls]
        qjs = [qj[sl] for sl in sls]
        ps = [(a - m_vec) * (c - m_vec) for a, c in zip(qis, qjs)]
        for sl, p in zip(sls, ps):
          ob[sl] = p
        return 0

      lax.fori_loop(0, chunk // (L * U), body, 0)

    start_in(0, 0)
    start_in(1, 1)

    def pipe_body(kk, _):
      for b in range(2):
        k_ = 2 * kk + b
        j = k_ - 1
        bj = 1 - b

        @pl.when(valid(k_))
        def _(k_=k_, b=b):
          wait_in(k_, b)
          start_gather(b)

        j_exists = valid(j) if b == 1 else jnp.logical_and(kk > 0, valid(j))

        @pl.when(j_exists)
        def _(k_=k_, j=j, b=b, bj=bj, kk=kk):
          wait_gather(bj)

          @pl.when(valid(k_ + 1))
          def _():
            start_in(k_ + 1, bj)

          out_busy = (kk >= 2) if b == 0 else (kk >= 1)

          @pl.when(out_busy)
          def _():
            wait_out(j - 2, bj)

          compute(bj)
          start_out(j, bj)
      return 0

    lax.fori_loop(0, k_steps // 2, pipe_body, 0)

    last = k_steps - 1
    bl = last % 2

    @pl.when(valid(last))
    def _():
      wait_gather(bl)
      wait_out(last - 2, bl)
      compute(bl)
      start_out(last, bl)

    for j in range(k_steps - 4, k_steps):
      @pl.when(valid(j) & jnp.logical_not(valid(j + 2)))
      def _(j=j):
        wait_out(j, j % 2)

  return k


@jax.jit
def kernel(element_idxs, neighbor_idxs, distances, base_charges):
  del distances
  b, n_atoms = element_idxs.shape
  n_pairs = neighbor_idxs.shape[1]
  elem = element_idxs.reshape(n_atoms).astype(jnp.int32)
  nbr = neighbor_idxs.astype(jnp.int32)
  base = jnp.zeros((L,), jnp.float32).at[:base_charges.shape[0]].set(
      base_charges.astype(jnp.float32))
  k = _make_kernel(n_atoms, n_pairs, chunk=6400)
  out = k(elem, nbr, base)
  return out.reshape(b, n_pairs)


# speedup vs baseline: 1.6854x; 1.6854x over previous
import functools

import jax
import jax.numpy as jnp
from jax import lax
from jax.experimental import pallas as pl
from jax.experimental.pallas import tpu as pltpu
from jax.experimental.pallas import tpu_sc as plsc

NC = 2
NS = 16
L = 16
NW = NC * NS
U = 5


def _make_kernel(n_atoms, n_pairs, chunk):
  n_chunks_total = n_pairs // chunk
  k_steps = -(-n_chunks_total // NW)
  k_steps += k_steps % 2
  slc = (n_atoms // NS) // (L * U) * (L * U)
  tail = n_atoms - NS * slc
  assert n_pairs % chunk == 0 and chunk % (L * U) == 0 and chunk % 128 == 0
  assert slc % 8 == 0 and tail % (L * U) == 0

  mesh = plsc.VectorSubcoreMesh(
      core_axis_name="c", subcore_axis_name="s",
      num_cores=NC, num_subcores=NS)

  @functools.partial(
      pl.kernel,
      out_type=jax.ShapeDtypeStruct((n_pairs,), jnp.float32),
      mesh=mesh,
      compiler_params=pltpu.CompilerParams(needs_layout_passes=False),
      scratch_types=[
          pltpu.VMEM((n_atoms,), jnp.float32),
          pltpu.VMEM_SHARED((n_atoms,), jnp.float32),
          pltpu.VMEM_SHARED((NS * L,), jnp.float32),
          pltpu.VMEM((L,), jnp.float32),
          pltpu.VMEM((chunk,), jnp.int32),
          pltpu.VMEM((chunk,), jnp.int32),
          pltpu.VMEM((chunk,), jnp.int32),
          pltpu.VMEM((chunk,), jnp.int32),
          pltpu.VMEM((chunk,), jnp.float32),
          pltpu.VMEM((chunk,), jnp.float32),
          pltpu.SemaphoreType.DMA((2,)),
          pltpu.SemaphoreType.DMA((2,)),
      ],
  )
  def k(elem_hbm, nbr_hbm, base_hbm, out_hbm,
        table_v, shared_q, shared_part, base_v,
        idx_i0, idx_i1, idx_j0, idx_j1, out0, out1,
        sem_in, sem_out):
    wid = lax.axis_index("s") * NC + lax.axis_index("c")
    sid = lax.axis_index("s")
    idx_i_b = (idx_i0, idx_i1)
    idx_j_b = (idx_j0, idx_j1)
    out_b = (out0, out1)

    pltpu.sync_copy(base_hbm, base_v)
    my_off = pl.multiple_of(sid * slc, 8)

    def convert(n):
      def body(i, acc):
        base_off = i * (L * U)
        sls = [pl.ds(base_off + u * L, L) for u in range(U)]
        es = [idx_i0[sl] for sl in sls]
        cs = [plsc.load_gather(base_v, [e]) for e in es]
        for sl, c in zip(sls, cs):
          out0[sl] = c
        return acc + sum(cs[1:], cs[0])
      return lax.fori_loop(0, n // (L * U), body,
                           jnp.zeros((L,), jnp.float32))

    pieces = []
    done = 0
    while done < slc:
      n = min(chunk, slc - done)
      pieces.append((done, n))
      done += n
    acc = jnp.zeros((L,), jnp.float32)
    for lo, n in pieces:
      pltpu.sync_copy(elem_hbm.at[pl.ds(my_off + lo, n)],
                      idx_i0.at[pl.ds(0, n)])
      acc = acc + convert(n)
      pltpu.sync_copy(out0.at[pl.ds(0, n)],
                      shared_q.at[pl.ds(my_off + lo, n)])

    @pl.when(sid == NS - 1)
    def _():
      pltpu.sync_copy(elem_hbm.at[pl.ds(NS * slc, tail)],
                      idx_i0.at[pl.ds(0, tail)])

    def tail_convert():
      a = convert(tail)
      pltpu.sync_copy(out0.at[pl.ds(0, tail)],
                      shared_q.at[pl.ds(NS * slc, tail)])
      return a

    acc = lax.cond(sid == NS - 1, lambda: acc + tail_convert(), lambda: acc)

    out0[pl.ds(0, L)] = acc
    pltpu.sync_copy(out0.at[pl.ds(0, L)],
                    shared_part.at[pl.ds(pl.multiple_of(sid * L, 8), L)])
    plsc.subcore_barrier()
    pltpu.sync_copy(shared_q, table_v)
    pltpu.sync_copy(shared_part, out1.at[pl.ds(0, NS * L)])
    tot = jnp.zeros((L,), jnp.float32)
    for t in range(NS):
      tot = tot + out1[pl.ds(t * L, L)]
    m = jnp.sum(tot) * (1.0 / float(n_atoms))
    m_vec = jnp.full((L,), m, jnp.float32)

    def chunk_off(k):
      g = wid + NW * k
      return pl.multiple_of(g * chunk, chunk)

    def valid(k):
      return wid + NW * k < n_chunks_total

    def start_in(k, b):
      off = chunk_off(k)
      pltpu.async_copy(nbr_hbm.at[0, pl.ds(off, chunk)], idx_i_b[b],
                       sem_in.at[b])
      pltpu.async_copy(nbr_hbm.at[1, pl.ds(off, chunk)], idx_j_b[b],
                       sem_in.at[b])

    def wait_in(k, b):
      off = chunk_off(k)
      pltpu.make_async_copy(nbr_hbm.at[0, pl.ds(off, chunk)], idx_i_b[b],
                            sem_in.at[b]).wait()
      pltpu.make_async_copy(nbr_hbm.at[1, pl.ds(off, chunk)], idx_j_b[b],
                            sem_in.at[b]).wait()

    def start_out(k, b):
      off = chunk_off(k)
      pltpu.async_copy(out_b[b], out_hbm.at[pl.ds(off, chunk)],
                       sem_out.at[b])

    def wait_out(k, b):
      off = chunk_off(k)
      pltpu.make_async_copy(out_b[b], out_hbm.at[pl.ds(off, chunk)],
                            sem_out.at[b]).wait()

    start_in(0, 0)
    start_in(1, 1)

    def chunk_pair_body(kk, _):
      for b in range(2):
        k = kk * 2 + b

        @pl.when(valid(k))
        def _(k=k, b=b):
          wait_in(k, b)

          @pl.when(kk > 0)
          def _():
            wait_out(k - 2, b)

          ib = idx_i_b[b]
          jb = idx_j_b[b]
          ob = out_b[b]

          def pair_body(t, _, ib=ib, jb=jb, ob=ob):
            base_t = t * (L * U)
            sls = [pl.ds(base_t + u * L, L) for u in range(U)]
            iis = [ib[sl] for sl in sls]
            jjs = [jb[sl] for sl in sls]
            qis = [plsc.load_gather(table_v, [x]) for x in iis]
            qjs = [plsc.load_gather(table_v, [x]) for x in jjs]
            ps = [(qi - m_vec) * (qj - m_vec) for qi, qj in zip(qis, qjs)]
            for sl, p in zip(sls, ps):
              ob[sl] = p
            return 0

          lax.fori_loop(0, chunk // (L * U), pair_body, 0)
          start_out(k, b)

          @pl.when(valid(k + 2))
          def _():
            start_in(k + 2, b)
      return 0

    lax.fori_loop(0, k_steps // 2, chunk_pair_body, 0)

    for k in range(max(0, k_steps - 4), k_steps):
      @pl.when(valid(k) & jnp.logical_not(valid(k + 2)))
      def _(k=k):
        wait_out(k, k % 2)

  return k


@jax.jit
def kernel(element_idxs, neighbor_idxs, distances, base_charges):
  del distances
  b, n_atoms = element_idxs.shape
  n_pairs = neighbor_idxs.shape[1]
  elem = element_idxs.reshape(n_atoms).astype(jnp.int32)
  nbr = neighbor_idxs.astype(jnp.int32)
  base = jnp.zeros((L,), jnp.float32).at[:base_charges.shape[0]].set(
      base_charges.astype(jnp.float32))
  k = _make_kernel(n_atoms, n_pairs, chunk=2560)
  out = k(elem, nbr, base)
  return out.reshape(b, n_pairs)
